# Initial kernel scaffold; baseline (speedup 1.0000x reference)
import jax
import jax.numpy as jnp
from jax.experimental import pallas as pl


def kernel(x, edge_index, W0, b0, W1, b1, W2, b2, W3, b3, W4, b4, W5, b5):
    raise NotImplementedError("write your pallas kernel here")


# speedup vs baseline: 7.1546x; 7.1546x over previous
import functools

import jax
import jax.numpy as jnp
from jax import lax
from jax.experimental import pallas as pl
from jax.experimental.pallas import tpu as pltpu
from jax.experimental.pallas import tpu_sc as plsc

_N = 10000
_NPAD = 10240
_E = 320000
_TRASH = _NPAD - 1
_NCORE = 2
_NSUB = 16
_NW = _NCORE * _NSUB
_EPT = _E // _NW
_EPT_PAD = 10240
_NB = _EPT_PAD // 128
_CH = 8
_NCH = _NB // _CH
_RPS = _NPAD // _NSUB


def _sc_agg(z, ridx, cidx, D):
    mesh = plsc.VectorSubcoreMesh(core_axis_name="c", subcore_axis_name="s")

    @functools.partial(
        pl.kernel,
        out_type=jax.ShapeDtypeStruct((_NCORE, _NPAD, D), jnp.float32),
        mesh=mesh,
        scratch_types=[
            pltpu.VMEM((_CH, 128), jnp.int32),
            pltpu.VMEM((_CH, 128), jnp.int32),
            pltpu.VMEM((128, D), jnp.float32),
            pltpu.VMEM((128, D), jnp.float32),
            pltpu.VMEM_SHARED((_NPAD, D), jnp.float32),
            pltpu.SemaphoreType.DMA,
            pltpu.SemaphoreType.DMA,
        ],
    )
    def agg(z_hbm, ridx_hbm, cidx_hbm, p_hbm,
            ridx_v, cidx_v, buf0, buf1, acc, sem0, sem1):
        c = lax.axis_index("c")
        s = lax.axis_index("s")
        wid = c * _NSUB + s
        base = s * _RPS

        def zrow(i, carry):
            for k in range(D // 16):
                buf0[i, pl.ds(k * 16, 16)] = jnp.zeros((16,), jnp.float32)
            return carry
        lax.fori_loop(0, 128, zrow, 0)

        def zcp(i, carry):
            pltpu.sync_copy(buf0, acc.at[pl.ds(base + i * 128, 128)])
            return carry
        lax.fori_loop(0, _RPS // 128, zcp, 0)
        plsc.subcore_barrier()

        bufs = (buf0, buf1)
        sems = (sem0, sem1)

        def chunk(cc, carry):
            pltpu.sync_copy(ridx_hbm.at[wid, pl.ds(cc * _CH, _CH)], ridx_v)
            pltpu.sync_copy(cidx_hbm.at[wid, pl.ds(cc * _CH, _CH)], cidx_v)
            pltpu.async_copy(z_hbm.at[ridx_v.at[0]], buf0, sem0)
            pltpu.async_copy(z_hbm.at[ridx_v.at[1]], buf1, sem1)
            for b in range(_CH):
                buf, sem = bufs[b % 2], sems[b % 2]
                pltpu.make_async_copy(z_hbm.at[ridx_v.at[b]], buf, sem).wait()
                pltpu.sync_copy(buf, acc.at[cidx_v.at[b]], add=True)
                if b + 2 < _CH:
                    pltpu.async_copy(z_hbm.at[ridx_v.at[b + 2]], buf, sem)
            return carry
        lax.fori_loop(0, _NCH, chunk, 0)
        plsc.subcore_barrier()
        pltpu.sync_copy(acc.at[pl.ds(base, _RPS)],
                        p_hbm.at[c, pl.ds(base, _RPS)])

    return agg(z, ridx, cidx)


def _sc_deg(cidx):
    D = 128
    mesh = plsc.VectorSubcoreMesh(core_axis_name="c", subcore_axis_name="s")

    @functools.partial(
        pl.kernel,
        out_type=jax.ShapeDtypeStruct((_NCORE, _NPAD, D), jnp.float32),
        mesh=mesh,
        scratch_types=[
            pltpu.VMEM((_CH, 128), jnp.int32),
            pltpu.VMEM((128, D), jnp.float32),
            pltpu.VMEM_SHARED((_NPAD, D), jnp.float32),
        ],
    )
    def deg(cidx_hbm, p_hbm, cidx_v, buf, acc):
        c = lax.axis_index("c")
        s = lax.axis_index("s")
        wid = c * _NSUB + s
        base = s * _RPS

        def fill(val):
            def frow(i, carry):
                for k in range(D // 16):
                    buf[i, pl.ds(k * 16, 16)] = jnp.full((16,), val, jnp.float32)
                return carry
            lax.fori_loop(0, 128, frow, 0)
        fill(0.0)

        def zcp(i, carry):
            pltpu.sync_copy(buf, acc.at[pl.ds(base + i * 128, 128)])
            return carry
        lax.fori_loop(0, _RPS // 128, zcp, 0)
        fill(1.0)
        plsc.subcore_barrier()

        def chunk(cc, carry):
            pltpu.sync_copy(cidx_hbm.at[wid, pl.ds(cc * _CH, _CH)], cidx_v)
            for b in range(_CH):
                pltpu.sync_copy(buf, acc.at[cidx_v.at[b]], add=True)
            return carry
        lax.fori_loop(0, _NCH, chunk, 0)
        plsc.subcore_barrier()
        pltpu.sync_copy(acc.at[pl.ds(base, _RPS)],
                        p_hbm.at[c, pl.ds(base, _RPS)])

    return deg(cidx)


_BLK = 1024


def _tc_first(x, w, dp0, dp1):
    def body(x_ref, w_ref, d0_ref, d1_ref, z_ref, dis_ref):
        deg = d0_ref[...] + d1_ref[...] + 1.0
        dis = lax.rsqrt(deg)
        dis_ref[...] = dis
        z_ref[...] = dis * jnp.dot(x_ref[...], w_ref[...],
                                   preferred_element_type=jnp.float32)
    blk = pl.BlockSpec((_BLK, 128), lambda i: (i, 0))
    wblk = pl.BlockSpec((128, 128), lambda i: (0, 0))
    return pl.pallas_call(
        body,
        grid=(_NPAD // _BLK,),
        in_specs=[blk, wblk, blk, blk],
        out_specs=[blk, blk],
        out_shape=[jax.ShapeDtypeStruct((_NPAD, 128), jnp.float32)] * 2,
    )(x, w, dp0, dp1)


def _tc_mid(p0, p1, z, dis, b, w, dn):
    def body(p0_ref, p1_ref, z_ref, dis_ref, disn_ref, b_ref, w_ref, out_ref):
        h = dis_ref[...] * (p0_ref[...] + p1_ref[...] + z_ref[...]) + b_ref[...]
        h = jnp.maximum(h, 0.0)
        out_ref[...] = disn_ref[...] * jnp.dot(h, w_ref[...],
                                               preferred_element_type=jnp.float32)
    blk = pl.BlockSpec((_BLK, 128), lambda i: (i, 0))
    blkn = pl.BlockSpec((_BLK, dn), lambda i: (i, 0))
    return pl.pallas_call(
        body,
        grid=(_NPAD // _BLK,),
        in_specs=[blk, blk, blk, blk, blkn,
                  pl.BlockSpec((1, 128), lambda i: (0, 0)),
                  pl.BlockSpec((128, dn), lambda i: (0, 0))],
        out_specs=blkn,
        out_shape=jax.ShapeDtypeStruct((_NPAD, dn), jnp.float32),
    )(p0, p1, z, dis, dis[:, :dn], b, w)


def _tc_final(p0, p1, z, dis, b):
    def body(p0_ref, p1_ref, z_ref, dis_ref, b_ref, out_ref):
        out_ref[...] = dis_ref[...] * (p0_ref[...] + p1_ref[...] + z_ref[...]) \
            + b_ref[...]
    blk = pl.BlockSpec((_BLK, 128), lambda i: (i, 0))
    return pl.pallas_call(
        body,
        grid=(_NPAD // _BLK,),
        in_specs=[blk, blk, blk, blk, pl.BlockSpec((1, 128), lambda i: (0, 0))],
        out_specs=blk,
        out_shape=jax.ShapeDtypeStruct((_NPAD, 128), jnp.float32),
    )(p0, p1, z, dis, b)


def kernel(x, edge_index, W0, b0, W1, b1, W2, b2, W3, b3, W4, b4, W5, b5):
    x = jnp.pad(x, ((0, _NPAD - _N), (0, 0)))
    row = edge_index[0].reshape(_NW, _EPT)
    col = edge_index[1].reshape(_NW, _EPT)
    pad_r = jnp.zeros((_NW, _EPT_PAD - _EPT), jnp.int32)
    pad_c = jnp.full((_NW, _EPT_PAD - _EPT), _TRASH, jnp.int32)
    ridx = jnp.concatenate([row, pad_r], axis=1).reshape(_NW, _NB, 128)
    cidx = jnp.concatenate([col, pad_c], axis=1).reshape(_NW, _NB, 128)

    dp = _sc_deg(cidx)
    z, dis = _tc_first(x, W0, dp[0], dp[1])

    Ws = [W1, W2, W3, W4]
    bs = [b0, b1, b2, b3]
    for i in range(4):
        p = _sc_agg(z, ridx, cidx, 128)
        z = _tc_mid(p[0], p[1], z, dis, bs[i].reshape(1, 128), Ws[i], 128)

    p = _sc_agg(z, ridx, cidx, 128)
    W5p = jnp.pad(W5, ((0, 0), (0, 124)))
    z = _tc_mid(p[0], p[1], z, dis, b4.reshape(1, 128), W5p, 128)

    p = _sc_agg(z, ridx, cidx, 128)
    b5p = jnp.pad(b5, (0, 124)).reshape(1, 128)
    out = _tc_final(p[0], p[1], z, dis, b5p)
    return out[:_N, :4]


# speedup vs baseline: 7.4613x; 1.0429x over previous
import functools

import jax
import jax.numpy as jnp
from jax import lax
from jax.experimental import pallas as pl
from jax.experimental.pallas import tpu as pltpu
from jax.experimental.pallas import tpu_sc as plsc

_N = 10000
_NPAD = 10240
_E = 320000
_TRASH = _NPAD - 1
_NCORE = 2
_NSUB = 16
_NW = _NCORE * _NSUB
_EPT = _E // _NW
_EPT_PAD = 10240
_NB = _EPT_PAD // 128
_CH = 40
_NCH = _NB // _CH
_RPS = _NPAD // _NSUB


def _sc_agg(z, ridx, cidx, D):
    mesh = plsc.VectorSubcoreMesh(core_axis_name="c", subcore_axis_name="s")

    @functools.partial(
        pl.kernel,
        out_type=jax.ShapeDtypeStruct((_NCORE, _NPAD, D), jnp.float32),
        mesh=mesh,
        scratch_types=[
            pltpu.VMEM((_CH, 128), jnp.int32),
            pltpu.VMEM((_CH, 128), jnp.int32),
            pltpu.VMEM((128, D), jnp.float32),
            pltpu.VMEM((128, D), jnp.float32),
            pltpu.VMEM_SHARED((_NPAD, D), jnp.float32),
            pltpu.SemaphoreType.DMA,
            pltpu.SemaphoreType.DMA,
        ],
    )
    def agg(z_hbm, ridx_hbm, cidx_hbm, p_hbm,
            ridx_v, cidx_v, buf0, buf1, acc, sem0, sem1):
        c = lax.axis_index("c")
        s = lax.axis_index("s")
        wid = c * _NSUB + s
        base = s * _RPS

        def zrow(i, carry):
            for k in range(D // 16):
                buf0[i, pl.ds(k * 16, 16)] = jnp.zeros((16,), jnp.float32)
            return carry
        lax.fori_loop(0, 128, zrow, 0)

        def zcp(i, carry):
            pltpu.sync_copy(buf0, acc.at[pl.ds(base + i * 128, 128)])
            return carry
        lax.fori_loop(0, _RPS // 128, zcp, 0)
        plsc.subcore_barrier()

        bufs = (buf0, buf1)
        sems = (sem0, sem1)

        def chunk(cc, carry):
            pltpu.sync_copy(ridx_hbm.at[wid, pl.ds(cc * _CH, _CH)], ridx_v)
            pltpu.sync_copy(cidx_hbm.at[wid, pl.ds(cc * _CH, _CH)], cidx_v)
            pltpu.async_copy(z_hbm.at[ridx_v.at[0]], buf0, sem0)
            pltpu.async_copy(z_hbm.at[ridx_v.at[1]], buf1, sem1)
            for b in range(_CH):
                buf, sem = bufs[b % 2], sems[b % 2]
                pltpu.make_async_copy(z_hbm.at[ridx_v.at[b]], buf, sem).wait()
                pltpu.sync_copy(buf, acc.at[cidx_v.at[b]], add=True)
                if b + 2 < _CH:
                    pltpu.async_copy(z_hbm.at[ridx_v.at[b + 2]], buf, sem)
            return carry
        lax.fori_loop(0, _NCH, chunk, 0)
        plsc.subcore_barrier()
        pltpu.sync_copy(acc.at[pl.ds(base, _RPS)],
                        p_hbm.at[c, pl.ds(base, _RPS)])

    return agg(z, ridx, cidx)


def _sc_deg(cidx):
    mesh = plsc.VectorSubcoreMesh(core_axis_name="c", subcore_axis_name="s")

    @functools.partial(
        pl.kernel,
        out_type=jax.ShapeDtypeStruct((_NCORE, _NPAD, 128), jnp.float32),
        mesh=mesh,
        scratch_types=[
            pltpu.VMEM((_NB, 128), jnp.int32),
            pltpu.VMEM((128, 128), jnp.float32),
            pltpu.VMEM_SHARED((_NPAD, 128), jnp.float32),
        ],
    )
    def deg(cidx_hbm, p_hbm, cidx_v, buf, acc):
        c = lax.axis_index("c")
        s = lax.axis_index("s")
        wid = c * _NSUB + s
        base = s * _RPS

        def zrow(i, carry):
            for k in range(8):
                buf[i, pl.ds(k * 16, 16)] = jnp.zeros((16,), jnp.float32)
            return carry
        lax.fori_loop(0, 128, zrow, 0)

        def zcp(i, carry):
            pltpu.sync_copy(buf, acc.at[pl.ds(base + i * 128, 128)])
            return carry
        lax.fori_loop(0, _RPS // 128, zcp, 0)
        plsc.subcore_barrier()

        def orow(i, carry):
            for k in range(8):
                buf[i, pl.ds(k * 16, 16)] = jnp.ones((16,), jnp.float32)
            return carry
        lax.fori_loop(0, 128, orow, 0)
        pltpu.sync_copy(cidx_hbm.at[wid], cidx_v)

        def body(b, carry):
            pltpu.sync_copy(buf, acc.at[cidx_v.at[b]], add=True)
            return carry
        lax.fori_loop(0, _NB, body, 0)
        plsc.subcore_barrier()
        pltpu.sync_copy(acc.at[pl.ds(base, _RPS)],
                        p_hbm.at[c, pl.ds(base, _RPS)])

    return deg(cidx)


_BLK = 1024


def _tc_first(x, w, dp):
    def body(x_ref, w_ref, dp0_ref, dp1_ref, z_ref, dis_ref):
        dis = lax.rsqrt(dp0_ref[...] + dp1_ref[...] + 1.0)
        dis_ref[...] = dis
        z_ref[...] = dis * jnp.dot(x_ref[...], w_ref[...],
                                   preferred_element_type=jnp.float32)
    blk = pl.BlockSpec((_BLK, 128), lambda i: (i, 0))
    wblk = pl.BlockSpec((128, 128), lambda i: (0, 0))
    return pl.pallas_call(
        body,
        grid=(_NPAD // _BLK,),
        in_specs=[blk, wblk, blk, blk],
        out_specs=[blk, blk],
        out_shape=[jax.ShapeDtypeStruct((_NPAD, 128), jnp.float32)] * 2,
    )(x, w, dp[0], dp[1])


def _tc_mid(p0, p1, z, dis, b, w, dn):
    def body(p0_ref, p1_ref, z_ref, dis_ref, disn_ref, b_ref, w_ref, out_ref):
        h = dis_ref[...] * (p0_ref[...] + p1_ref[...] + z_ref[...]) + b_ref[...]
        h = jnp.maximum(h, 0.0)
        out_ref[...] = disn_ref[...] * jnp.dot(h, w_ref[...],
                                               preferred_element_type=jnp.float32)
    blk = pl.BlockSpec((_BLK, 128), lambda i: (i, 0))
    blkn = pl.BlockSpec((_BLK, dn), lambda i: (i, 0))
    return pl.pallas_call(
        body,
        grid=(_NPAD // _BLK,),
        in_specs=[blk, blk, blk, blk, blkn,
                  pl.BlockSpec((1, 128), lambda i: (0, 0)),
                  pl.BlockSpec((128, dn), lambda i: (0, 0))],
        out_specs=blkn,
        out_shape=jax.ShapeDtypeStruct((_NPAD, dn), jnp.float32),
    )(p0, p1, z, dis, dis[:, :dn], b, w)


def _tc_final(p0, p1, z, dis, b):
    def body(p0_ref, p1_ref, z_ref, dis_ref, b_ref, out_ref):
        out_ref[...] = dis_ref[...] * (p0_ref[...] + p1_ref[...] + z_ref[...]) \
            + b_ref[...]
    blk = pl.BlockSpec((_BLK, 128), lambda i: (i, 0))
    return pl.pallas_call(
        body,
        grid=(_NPAD // _BLK,),
        in_specs=[blk, blk, blk, blk, pl.BlockSpec((1, 128), lambda i: (0, 0))],
        out_specs=blk,
        out_shape=jax.ShapeDtypeStruct((_NPAD, 128), jnp.float32),
    )(p0, p1, z, dis, b)


def kernel(x, edge_index, W0, b0, W1, b1, W2, b2, W3, b3, W4, b4, W5, b5):
    x = jnp.pad(x, ((0, _NPAD - _N), (0, 0)))
    row = edge_index[0].reshape(_NW, _EPT)
    col = edge_index[1].reshape(_NW, _EPT)
    pad_r = jnp.zeros((_NW, _EPT_PAD - _EPT), jnp.int32)
    pad_c = jnp.full((_NW, _EPT_PAD - _EPT), _TRASH, jnp.int32)
    ridx = jnp.concatenate([row, pad_r], axis=1).reshape(_NW, _NB, 128)
    cidx = jnp.concatenate([col, pad_c], axis=1).reshape(_NW, _NB, 128)

    dp = _sc_deg(cidx)
    z, dis = _tc_first(x, W0, dp)

    Ws = [W1, W2, W3, W4]
    bs = [b0, b1, b2, b3]
    for i in range(4):
        p = _sc_agg(z, ridx, cidx, 128)
        z = _tc_mid(p[0], p[1], z, dis, bs[i].reshape(1, 128), Ws[i], 128)

    p = _sc_agg(z, ridx, cidx, 128)
    W5p = jnp.pad(W5, ((0, 0), (0, 124)))
    z = _tc_mid(p[0], p[1], z, dis, b4.reshape(1, 128), W5p, 128)

    p = _sc_agg(z, ridx, cidx, 128)
    b5p = jnp.pad(b5, (0, 124)).reshape(1, 128)
    out = _tc_final(p[0], p[1], z, dis, b5p)
    return out[:_N, :4]
